# baseline (speedup 1.0000x reference)
import numpy as np
import jax
import jax.numpy as jnp
from jax import lax
from jax.experimental import pallas as pl
from jax.experimental.pallas import tpu as pltpu
from jax.experimental.pallas import tpu_sc as plsc

_N_LEVELS = 16
_TABLE = 1 << 19
_MASK = _TABLE - 1
_BATCH = 262144
_P2 = int(np.uint32(2654435761).view(np.int32))
_P3 = int(np.uint32(805459861).view(np.int32))
_BF = float(np.exp((np.log(512.0) - np.log(16.0)) / 15))
_RES = [float(np.floor(16.0 * (_BF ** i))) for i in range(_N_LEVELS)]

_NC, _NS = 2, 16
_NW = _NC * _NS
_PW = _BATCH // _NW
_CHUNK = 256
_NCHUNK = _PW // _CHUNK
_NIDX = 8 * _CHUNK
_NBLK = _N_LEVELS * _TABLE // 128
_BPW = _NBLK // _NW
_FB = 64
_TROWS = _N_LEVELS * _TABLE // 8


def _fmt_body(src, dst, in_v, out_v):
    wid = lax.axis_index("s") * _NC + lax.axis_index("c")
    lanes = lax.iota(jnp.int32, 16)
    dei = (lanes >> 1) + ((lanes & 1) << 7)

    def batch(bi, carry):
        blk0 = wid * _BPW + bi * _FB
        pltpu.sync_copy(src.at[pl.ds(blk0, _FB)], in_v)

        def pb(b, c):
            rowv = jnp.zeros((16,), jnp.int32) + b
            for m in range(16):
                v = plsc.load_gather(in_v, [rowv, dei + (m * 8)])
                out_v[b * 16 + m, :] = v
            return c

        lax.fori_loop(0, _FB, pb, 0)
        pltpu.sync_copy(out_v, dst.at[pl.ds(blk0 * 16, _FB * 16)])
        return carry

    lax.fori_loop(0, _BPW // _FB, batch, 0)


def _body(xs, ys, zs, tab, out, x_v, y_v, z_v, w_v, idx_v, low_v, rows_v,
          out_v, sem0, sem1):
    wid = lax.axis_index("s") * _NC + lax.axis_index("c")
    lanes = lax.iota(jnp.int32, 16)
    sems = (sem0, sem1)

    def hash_level(l, buf, co):
        r = jnp.float32(_RES[l])
        loff = jnp.int32(l * (_TABLE // 8))

        @plsc.parallel_loop(0, _CHUNK // 16, unroll=2)
        def hb(i):
            p = i * 16
            x = x_v[pl.ds(co + p, 16)] * r
            y = y_v[pl.ds(co + p, 16)] * r
            z = z_v[pl.ds(co + p, 16)] * r
            xi = x.astype(jnp.int32)
            yi = y.astype(jnp.int32)
            zi = z.astype(jnp.int32)
            w_v[buf, 0, pl.ds(p, 16)] = x - xi.astype(jnp.float32)
            w_v[buf, 1, pl.ds(p, 16)] = y - yi.astype(jnp.float32)
            w_v[buf, 2, pl.ds(p, 16)] = z - zi.astype(jnp.float32)
            b0 = yi * _P2
            b1 = b0 + _P2
            c0 = zi * _P3
            c1 = c0 + _P3
            x1 = xi + 1
            e00 = xi ^ b0
            e01 = xi ^ b1
            e10 = x1 ^ b0
            e11 = x1 ^ b1
            corners = ((e00, c0), (e00, c1), (e01, c0), (e01, c1),
                       (e10, c0), (e10, c1), (e11, c0), (e11, c1))
            for j, (e, cc) in enumerate(corners):
                h = (e ^ cc) & _MASK
                idx_v[buf, pl.ds(j * _CHUNK + p, 16)] = (h >> 3) + loff
                low_v[buf, pl.ds(j * _CHUNK + p, 16)] = h & 7

        pltpu.async_copy(tab.at[idx_v.at[buf]], rows_v.at[buf], sems[buf])

    def drain_level(buf):
        pltpu.make_async_copy(tab.at[idx_v.at[buf]], rows_v.at[buf],
                              sems[buf]).wait()

    def interp_level(l, buf):
        rows = rows_v.at[buf]

        @plsc.parallel_loop(0, _CHUNK // 16, unroll=2)
        def ib(i):
            p = i * 16
            wx = w_v[buf, 0, pl.ds(p, 16)]
            wy = w_v[buf, 1, pl.ds(p, 16)]
            wz = w_v[buf, 2, pl.ds(p, 16)]
            ux = 1.0 - wx
            uy = 1.0 - wy
            uz = 1.0 - wz
            w00 = ux * uy
            w01 = ux * wy
            w10 = wx * uy
            w11 = wx * wy
            wj = (w00 * uz, w00 * wz, w01 * uz, w01 * wz,
                  w10 * uz, w10 * wz, w11 * uz, w11 * wz)
            acc0 = jnp.zeros((16,), jnp.float32)
            acc1 = jnp.zeros((16,), jnp.float32)
            for j in range(8):
                ridx = lanes + (j * _CHUNK + p)
                lv = low_v[buf, pl.ds(j * _CHUNK + p, 16)]
                fcol0 = lv + lv
                fcol1 = fcol0 + 1
                v0 = plsc.load_gather(rows, [ridx, fcol0])
                v1 = plsc.load_gather(rows, [ridx, fcol1])
                acc0 = acc0 + wj[j] * v0
                acc1 = acc1 + wj[j] * v1
            out_v[2 * l, pl.ds(p, 16)] = acc0
            out_v[2 * l + 1, pl.ds(p, 16)] = acc1

    wbase = wid * _PW
    pltpu.sync_copy(xs.at[pl.ds(wbase, _PW)], x_v)
    pltpu.sync_copy(ys.at[pl.ds(wbase, _PW)], y_v)
    pltpu.sync_copy(zs.at[pl.ds(wbase, _PW)], z_v)

    def chunk_body(ci, carry):
        co = ci * _CHUNK
        hash_level(0, 0, co)
        for l in range(1, _N_LEVELS):
            hash_level(l, l % 2, co)
            drain_level((l - 1) % 2)
            interp_level(l - 1, (l - 1) % 2)
        drain_level((_N_LEVELS - 1) % 2)
        interp_level(_N_LEVELS - 1, (_N_LEVELS - 1) % 2)
        pltpu.sync_copy(out_v, out.at[:, pl.ds(wbase + co, _CHUNK)])
        return carry

    lax.fori_loop(0, _NCHUNK, chunk_body, 0)


@jax.jit
def kernel(input_points, tables):
    xs = input_points[:, 0]
    ys = input_points[:, 1]
    zs = input_points[:, 2]
    tabn = (tables.reshape(_N_LEVELS, _TABLE // 128, 128, 2)
            .transpose(0, 1, 3, 2)
            .reshape(_NBLK, 256))
    mesh = plsc.VectorSubcoreMesh(core_axis_name="c", subcore_axis_name="s",
                                  num_cores=_NC, num_subcores=_NS)
    cp = pltpu.CompilerParams(
        use_tc_tiling_on_sc=False, needs_layout_passes=False,
        disable_bounds_checks=True)
    fmt = pl.kernel(
        _fmt_body,
        out_type=jax.ShapeDtypeStruct((_TROWS, 16), jnp.float32),
        mesh=mesh,
        compiler_params=cp,
        scratch_types=[
            pltpu.VMEM((_FB, 256), jnp.float32),
            pltpu.VMEM((_FB * 16, 16), jnp.float32),
        ],
    )
    tab = fmt(tabn)
    f = pl.kernel(
        _body,
        out_type=jax.ShapeDtypeStruct((32, _BATCH), jnp.float32),
        mesh=mesh,
        compiler_params=cp,
        scratch_types=[
            pltpu.VMEM((_PW,), jnp.float32),
            pltpu.VMEM((_PW,), jnp.float32),
            pltpu.VMEM((_PW,), jnp.float32),
            pltpu.VMEM((2, 3, _CHUNK), jnp.float32),
            pltpu.VMEM((2, _NIDX), jnp.int32),
            pltpu.VMEM((2, _NIDX), jnp.int32),
            pltpu.VMEM((2, _NIDX, 16), jnp.float32),
            pltpu.VMEM((32, _CHUNK), jnp.float32),
            pltpu.SemaphoreType.DMA,
            pltpu.SemaphoreType.DMA,
        ],
    )
    return jnp.transpose(f(xs, ys, zs, tab))


# speedup vs baseline: 1.0053x; 1.0053x over previous
import numpy as np
import jax
import jax.numpy as jnp
from jax import lax
from jax.experimental import pallas as pl
from jax.experimental.pallas import tpu as pltpu
from jax.experimental.pallas import tpu_sc as plsc

_N_LEVELS = 16
_TABLE = 1 << 19
_MASK = _TABLE - 1
_BATCH = 262144
_P2 = int(np.uint32(2654435761).view(np.int32))
_P3 = int(np.uint32(805459861).view(np.int32))
_BF = float(np.exp((np.log(512.0) - np.log(16.0)) / 15))
_RES = [float(np.floor(16.0 * (_BF ** i))) for i in range(_N_LEVELS)]

_NC, _NS = 2, 16
_NW = _NC * _NS
_PW = _BATCH // _NW
_CHUNK = 256
_NCHUNK = _PW // _CHUNK
_NIDX = 8 * _CHUNK
_NBLK = _N_LEVELS * _TABLE // 128
_BPW = _NBLK // _NW
_FB = 64
_TROWS = _N_LEVELS * _TABLE // 8


def _fmt_body(src, dst, in_v, out_v):
    wid = lax.axis_index("s") * _NC + lax.axis_index("c")
    lanes = lax.iota(jnp.int32, 16)
    dei = (lanes >> 1) + ((lanes & 1) << 7)

    def batch(bi, carry):
        blk0 = wid * _BPW + bi * _FB
        pltpu.sync_copy(src.at[pl.ds(blk0, _FB)], in_v)

        def pb(b, c):
            rowv = jnp.zeros((16,), jnp.int32) + b
            for m in range(16):
                v = plsc.load_gather(in_v, [rowv, dei + (m * 8)])
                out_v[b * 16 + m, :] = v
            return c

        lax.fori_loop(0, _FB, pb, 0)
        pltpu.sync_copy(out_v, dst.at[pl.ds(blk0 * 16, _FB * 16)])
        return carry

    lax.fori_loop(0, _BPW // _FB, batch, 0)


def _body(xs, ys, zs, tab, out, x_v, y_v, z_v, w_v, idx_v, low_v, rows_v,
          out_v, sem0, sem1):
    wid = lax.axis_index("s") * _NC + lax.axis_index("c")
    lanes = lax.iota(jnp.int32, 16)
    sems = (sem0, sem1)

    def hash_level(l, buf, co):
        r = jnp.float32(_RES[l])
        loff = jnp.int32(l * (_TABLE // 8))

        def hb(i, c):
            p = i * 16
            x = x_v[pl.ds(co + p, 16)] * r
            y = y_v[pl.ds(co + p, 16)] * r
            z = z_v[pl.ds(co + p, 16)] * r
            xi = x.astype(jnp.int32)
            yi = y.astype(jnp.int32)
            zi = z.astype(jnp.int32)
            w_v[buf, 0, pl.ds(p, 16)] = x - xi.astype(jnp.float32)
            w_v[buf, 1, pl.ds(p, 16)] = y - yi.astype(jnp.float32)
            w_v[buf, 2, pl.ds(p, 16)] = z - zi.astype(jnp.float32)
            b0 = yi * _P2
            b1 = b0 + _P2
            c0 = zi * _P3
            c1 = c0 + _P3
            x1 = xi + 1
            e00 = xi ^ b0
            e01 = xi ^ b1
            e10 = x1 ^ b0
            e11 = x1 ^ b1
            corners = ((e00, c0), (e00, c1), (e01, c0), (e01, c1),
                       (e10, c0), (e10, c1), (e11, c0), (e11, c1))
            for j, (e, cc) in enumerate(corners):
                h = (e ^ cc) & _MASK
                idx_v[buf, pl.ds(j * _CHUNK + p, 16)] = (h >> 3) + loff
                low_v[buf, pl.ds(j * _CHUNK + p, 16)] = h & 7
            return c

        lax.fori_loop(0, _CHUNK // 16, hb, 0)

        pltpu.async_copy(tab.at[idx_v.at[buf]], rows_v.at[buf], sems[buf])

    def drain_level(buf):
        pltpu.make_async_copy(tab.at[idx_v.at[buf]], rows_v.at[buf],
                              sems[buf]).wait()

    def interp_level(l, buf):
        rows = rows_v.at[buf]

        def ib(i, c):
            p = i * 16
            wx = w_v[buf, 0, pl.ds(p, 16)]
            wy = w_v[buf, 1, pl.ds(p, 16)]
            wz = w_v[buf, 2, pl.ds(p, 16)]
            ux = 1.0 - wx
            uy = 1.0 - wy
            uz = 1.0 - wz
            w00 = ux * uy
            w01 = ux * wy
            w10 = wx * uy
            w11 = wx * wy
            wj = (w00 * uz, w00 * wz, w01 * uz, w01 * wz,
                  w10 * uz, w10 * wz, w11 * uz, w11 * wz)
            acc0 = jnp.zeros((16,), jnp.float32)
            acc1 = jnp.zeros((16,), jnp.float32)
            for j in range(8):
                ridx = lanes + (j * _CHUNK + p)
                lv = low_v[buf, pl.ds(j * _CHUNK + p, 16)]
                fcol0 = lv + lv
                fcol1 = fcol0 + 1
                v0 = plsc.load_gather(rows, [ridx, fcol0])
                v1 = plsc.load_gather(rows, [ridx, fcol1])
                acc0 = acc0 + wj[j] * v0
                acc1 = acc1 + wj[j] * v1
            out_v[2 * l, pl.ds(p, 16)] = acc0
            out_v[2 * l + 1, pl.ds(p, 16)] = acc1
            return c

        lax.fori_loop(0, _CHUNK // 16, ib, 0)

    wbase = wid * _PW
    pltpu.sync_copy(xs.at[pl.ds(wbase, _PW)], x_v)
    pltpu.sync_copy(ys.at[pl.ds(wbase, _PW)], y_v)
    pltpu.sync_copy(zs.at[pl.ds(wbase, _PW)], z_v)

    def chunk_body(ci, carry):
        co = ci * _CHUNK
        hash_level(0, 0, co)
        for l in range(1, _N_LEVELS):
            hash_level(l, l % 2, co)
            drain_level((l - 1) % 2)
            interp_level(l - 1, (l - 1) % 2)
        drain_level((_N_LEVELS - 1) % 2)
        interp_level(_N_LEVELS - 1, (_N_LEVELS - 1) % 2)
        pltpu.sync_copy(out_v, out.at[:, pl.ds(wbase + co, _CHUNK)])
        return carry

    lax.fori_loop(0, _NCHUNK, chunk_body, 0)


@jax.jit
def kernel(input_points, tables):
    xs = input_points[:, 0]
    ys = input_points[:, 1]
    zs = input_points[:, 2]
    tabn = (tables.reshape(_N_LEVELS, _TABLE // 128, 128, 2)
            .transpose(0, 1, 3, 2)
            .reshape(_NBLK, 256))
    mesh = plsc.VectorSubcoreMesh(core_axis_name="c", subcore_axis_name="s",
                                  num_cores=_NC, num_subcores=_NS)
    cp = pltpu.CompilerParams(
        use_tc_tiling_on_sc=False, needs_layout_passes=False,
        disable_bounds_checks=True)
    fmt = pl.kernel(
        _fmt_body,
        out_type=jax.ShapeDtypeStruct((_TROWS, 16), jnp.float32),
        mesh=mesh,
        compiler_params=cp,
        scratch_types=[
            pltpu.VMEM((_FB, 256), jnp.float32),
            pltpu.VMEM((_FB * 16, 16), jnp.float32),
        ],
    )
    tab = fmt(tabn)
    f = pl.kernel(
        _body,
        out_type=jax.ShapeDtypeStruct((32, _BATCH), jnp.float32),
        mesh=mesh,
        compiler_params=cp,
        scratch_types=[
            pltpu.VMEM((_PW,), jnp.float32),
            pltpu.VMEM((_PW,), jnp.float32),
            pltpu.VMEM((_PW,), jnp.float32),
            pltpu.VMEM((2, 3, _CHUNK), jnp.float32),
            pltpu.VMEM((2, _NIDX), jnp.int32),
            pltpu.VMEM((2, _NIDX), jnp.int32),
            pltpu.VMEM((2, _NIDX, 16), jnp.float32),
            pltpu.VMEM((32, _CHUNK), jnp.float32),
            pltpu.SemaphoreType.DMA,
            pltpu.SemaphoreType.DMA,
        ],
    )
    return jnp.transpose(f(xs, ys, zs, tab))
